# Initial kernel scaffold; baseline (speedup 1.0000x reference)
import functools

import jax
import jax.numpy as jnp
from jax import lax
from jax.experimental import pallas as pl
from jax.experimental.pallas import tpu as pltpu
from jax.experimental.pallas import tpu_sc as plsc

_D = 32
_CHUNK = 128
_GPB = 8
_BLK = _CHUNK * _GPB


@functools.cache
def _make_kernel(N: int):
    info = plsc.get_sparse_core_info()
    nc, ns = info.num_cores, info.num_subcores
    nw = nc * ns
    n_w = N // nw
    n_blk = n_w // _BLK
    mesh = plsc.VectorSubcoreMesh(core_axis_name="c", subcore_axis_name="s")

    @functools.partial(
        pl.kernel,
        mesh=mesh,
        out_type=jax.ShapeDtypeStruct((N, _D), jnp.float32),
        scratch_types=[
            pltpu.VMEM((_GPB, _CHUNK), jnp.int32),
            pltpu.VMEM((_BLK, _D), jnp.float32),
            pltpu.SemaphoreType.DMA,
        ],
    )
    def k(idx_hbm, table_hbm, out_hbm, idx_v, rows_v, sem):
        wid = lax.axis_index("s") * nc + lax.axis_index("c")
        base = wid * n_w

        def step(b, carry):
            off = base + b * _BLK
            pltpu.sync_copy(idx_hbm.at[pl.ds(off // _CHUNK, _GPB)], idx_v)
            copies = []
            for j in range(_GPB):
                copies.append(pltpu.async_copy(
                    table_hbm.at[idx_v.at[j]],
                    rows_v.at[pl.ds(j * _CHUNK, _CHUNK)],
                    sem))
            for cp in copies:
                cp.wait()
            pltpu.sync_copy(rows_v, out_hbm.at[pl.ds(off, _BLK)])
            return carry

        lax.fori_loop(0, n_blk, step, 0)

    return k


def kernel(batch, table):
    B, H = batch.shape
    N = B * H
    idx = batch.reshape(N // _CHUNK, _CHUNK).astype(jnp.int32)
    out = _make_kernel(N)(idx, table)
    return out.reshape(B, H, _D)


# speedup vs baseline: 4.8099x; 4.8099x over previous
import functools

import jax
import jax.numpy as jnp
from jax import lax
from jax.experimental import pallas as pl
from jax.experimental.pallas import tpu as pltpu
from jax.experimental.pallas import tpu_sc as plsc

_D = 32
_CHUNK = 128
_GPB = 8
_BLK = _CHUNK * _GPB


@functools.cache
def _make_kernel(N: int):
    info = plsc.get_sparse_core_info()
    nc, ns = info.num_cores, info.num_subcores
    nw = nc * ns
    n_w = N // nw
    n_blk = n_w // _BLK
    mesh = plsc.VectorSubcoreMesh(core_axis_name="c", subcore_axis_name="s")

    @functools.partial(
        pl.kernel,
        mesh=mesh,
        out_type=jax.ShapeDtypeStruct((N, _D), jnp.float32),
        scratch_types=[
            pltpu.VMEM((_GPB, _CHUNK), jnp.int32),
            pltpu.VMEM((_BLK, _D), jnp.float32),
            pltpu.SemaphoreType.DMA,
        ],
        compiler_params=pltpu.CompilerParams(use_tc_tiling_on_sc=False),
    )
    def k(idx_hbm, table_hbm, out_hbm, idx_v, rows_v, sem):
        wid = lax.axis_index("s") * nc + lax.axis_index("c")
        base = wid * n_w

        def step(b, carry):
            off = pl.multiple_of(base + b * _BLK, _BLK)
            irow = pl.multiple_of((base + b * _BLK) // _CHUNK, _GPB)
            pltpu.sync_copy(idx_hbm.at[pl.ds(irow, _GPB)], idx_v)
            copies = []
            for j in range(_GPB):
                copies.append(pltpu.async_copy(
                    table_hbm.at[idx_v.at[j]],
                    rows_v.at[pl.ds(j * _CHUNK, _CHUNK)],
                    sem))
            for cp in copies:
                cp.wait()
            pltpu.sync_copy(rows_v, out_hbm.at[pl.ds(off, _BLK)])
            return carry

        lax.fori_loop(0, n_blk, step, 0)

    return k


def kernel(batch, table):
    B, H = batch.shape
    N = B * H
    idx = batch.reshape(N // _CHUNK, _CHUNK).astype(jnp.int32)
    out = _make_kernel(N)(idx, table)
    return out.reshape(B, H, _D)


# speedup vs baseline: 4.9585x; 1.0309x over previous
import functools

import jax
import jax.numpy as jnp
from jax import lax
from jax.experimental import pallas as pl
from jax.experimental.pallas import tpu as pltpu
from jax.experimental.pallas import tpu_sc as plsc

_D = 32
_CHUNK = 128
_GPB = 8
_BLK = _CHUNK * _GPB


@functools.cache
def _make_kernel(N: int):
    info = plsc.get_sparse_core_info()
    nc, ns = info.num_cores, info.num_subcores
    nw = nc * ns
    n_w = N // nw
    n_blk = n_w // _BLK
    assert n_blk % 2 == 0 and n_blk >= 4
    mesh = plsc.VectorSubcoreMesh(core_axis_name="c", subcore_axis_name="s")

    @functools.partial(
        pl.kernel,
        mesh=mesh,
        out_type=jax.ShapeDtypeStruct((N, _D), jnp.float32),
        scratch_types=[
            pltpu.VMEM((2, _GPB, _CHUNK), jnp.int32),
            pltpu.VMEM((2, _BLK, _D), jnp.float32),
            pltpu.SemaphoreType.DMA,
            pltpu.SemaphoreType.DMA,
            pltpu.SemaphoreType.DMA,
            pltpu.SemaphoreType.DMA,
        ],
        compiler_params=pltpu.CompilerParams(use_tc_tiling_on_sc=False),
    )
    def k(idx_hbm, table_hbm, out_hbm, idx_v, rows_v, sg0, sg1, sw0, sw1):
        sem_g = (sg0, sg1)
        sem_w = (sw0, sw1)
        wid = lax.axis_index("s") * nc + lax.axis_index("c")
        base = wid * n_w

        def fire_block(b, p):
            irow = pl.multiple_of((base + b * _BLK) // _CHUNK, _GPB)
            pltpu.sync_copy(idx_hbm.at[pl.ds(irow, _GPB)], idx_v.at[p])
            for j in range(_GPB):
                pltpu.async_copy(
                    table_hbm.at[idx_v.at[p, j]],
                    rows_v.at[p, pl.ds(j * _CHUNK, _CHUNK)],
                    sem_g[p])

        def wait_gathers(p):
            pltpu.make_async_copy(
                table_hbm.at[pl.ds(0, _BLK)], rows_v.at[p], sem_g[p]).wait()

        def fire_wb(b, p):
            off = pl.multiple_of(base + b * _BLK, _BLK)
            pltpu.async_copy(rows_v.at[p], out_hbm.at[pl.ds(off, _BLK)],
                             sem_w[p])

        def wait_wb(p):
            pltpu.make_async_copy(
                rows_v.at[p], out_hbm.at[pl.ds(0, _BLK)], sem_w[p]).wait()

        fire_block(0, 0)
        fire_block(1, 1)
        wait_gathers(0)
        fire_wb(0, 0)

        def step(i, carry):
            g = 2 * i
            for p in (0, 1):
                b = g + p
                wait_wb(p)
                fire_block(b, p)
                wait_gathers(1 - p)
                fire_wb(b - 1, 1 - p)
            return carry

        lax.fori_loop(1, n_blk // 2, step, 0)

        wait_gathers(1)
        fire_wb(n_blk - 1, 1)
        wait_wb(0)
        wait_wb(1)

    return k


def kernel(batch, table):
    B, H = batch.shape
    N = B * H
    idx = batch.reshape(N // _CHUNK, _CHUNK).astype(jnp.int32)
    out = _make_kernel(N)(idx, table)
    return out.reshape(B, H, _D)


# speedup vs baseline: 5.6373x; 1.1369x over previous
import functools

import jax
import jax.numpy as jnp
from jax import lax
from jax.experimental import pallas as pl
from jax.experimental.pallas import tpu as pltpu
from jax.experimental.pallas import tpu_sc as plsc

_D = 32
_CHUNK = 128
_GPB = 8
_BLK = _CHUNK * _GPB


@functools.cache
def _make_kernel(N: int):
    info = plsc.get_sparse_core_info()
    nc, ns = info.num_cores, info.num_subcores
    nw = nc * ns
    n_w = N // nw
    n_blk = n_w // _BLK
    assert n_blk % 2 == 0 and n_blk >= 4
    mesh = plsc.VectorSubcoreMesh(core_axis_name="c", subcore_axis_name="s")

    @functools.partial(
        pl.kernel,
        mesh=mesh,
        out_type=jax.ShapeDtypeStruct((N, _D), jnp.float32),
        scratch_types=[
            pltpu.VMEM((2, _GPB, _CHUNK), jnp.int32),
            pltpu.VMEM((2, _BLK, _D), jnp.float32),
            pltpu.SemaphoreType.DMA,
            pltpu.SemaphoreType.DMA,
            pltpu.SemaphoreType.DMA,
            pltpu.SemaphoreType.DMA,
        ],
        compiler_params=pltpu.CompilerParams(use_tc_tiling_on_sc=False),
    )
    def k(idx_hbm, table_hbm, out_hbm, idx_v, rows_v, sg0, sg1, sw0, sw1):
        sem_g = (sg0, sg1)
        sem_w = (sw0, sw1)
        wid = lax.axis_index("s") * nc + lax.axis_index("c")
        base = wid * n_w

        def fire_block(b, p):
            irow = pl.multiple_of((base + b * _BLK) // _CHUNK, _GPB)
            pltpu.sync_copy(idx_hbm.at[pl.ds(irow, _GPB)], idx_v.at[p])
            for j in range(_GPB):
                pltpu.async_copy(
                    table_hbm.at[idx_v.at[p, j]],
                    rows_v.at[p, pl.ds(j * _CHUNK, _CHUNK)],
                    sem_g[p])

        def wait_gathers(p):
            pltpu.make_async_copy(
                table_hbm.at[pl.ds(0, _BLK)], rows_v.at[p], sem_g[p]).wait()

        def fire_wb(b, p):
            off = pl.multiple_of(base + b * _BLK, _BLK)
            pltpu.async_copy(rows_v.at[p], out_hbm.at[pl.ds(off, _BLK)],
                             sem_w[p])

        def wait_wb(p):
            pltpu.make_async_copy(
                rows_v.at[p], out_hbm.at[pl.ds(0, _BLK)], sem_w[p]).wait()

        fire_block(0, 0)
        fire_block(1, 1)
        wait_gathers(0)
        fire_wb(0, 0)

        def step(i, carry):
            g = 2 * i
            for p in (0, 1):
                b = g + p
                wait_wb(p)
                fire_block(b, p)
                wait_gathers(1 - p)
                fire_wb(b - 1, 1 - p)
            return carry

        lax.fori_loop(1, n_blk // 2, step, 0)

        wait_gathers(1)
        fire_wb(n_blk - 1, 1)
        wait_wb(0)
        wait_wb(1)

    return k


@functools.cache
def _make_transpose(B: int, H: int):
    nb = B // 4

    def body(x_ref, o_ref):
        xt = x_ref[0].T
        for j in range(4):
            o_ref[0, :, j * nb:(j + 1) * nb] = xt[_D * j:_D * (j + 1), :]

    return pl.pallas_call(
        body,
        grid=(H,),
        in_specs=[pl.BlockSpec((1, nb, 128), lambda h: (h, 0, 0))],
        out_specs=pl.BlockSpec((1, _D, B), lambda h: (h, 0, 0)),
        out_shape=jax.ShapeDtypeStruct((H, _D, B), jnp.float32),
    )


def kernel(batch, table):
    B, H = batch.shape
    N = B * H
    idx = (jnp.transpose(batch)
           .reshape(H, 4, B // 4)
           .transpose(0, 2, 1)
           .reshape(N // _CHUNK, _CHUNK)
           .astype(jnp.int32))
    g = _make_kernel(N)(idx, table)
    g3 = g.reshape(H, B // 4, 128)
    out_t = _make_transpose(B, H)(g3)
    return jnp.transpose(out_t, (2, 0, 1))


# speedup vs baseline: 11.7933x; 2.0920x over previous
import functools

import jax
import jax.numpy as jnp
from jax import lax
from jax.experimental import pallas as pl
from jax.experimental.pallas import tpu as pltpu
from jax.experimental.pallas import tpu_sc as plsc

_D = 32
_CHUNK = 128
_GPB = 8
_BLK = _CHUNK * _GPB


@functools.cache
def _make_detile(V: int):
    lanes = 4096
    grid = (V + lanes - 1) // lanes

    def body(x_ref, o_ref):
        y = x_ref[...].T.reshape(lanes // 4, 4, _D)
        for j in range(4):
            o_ref[:, _D * j:_D * (j + 1)] = y[:, j, :]

    return pl.pallas_call(
        body,
        grid=(grid,),
        in_specs=[pl.BlockSpec((_D, lanes), lambda i: (0, i))],
        out_specs=pl.BlockSpec((lanes // 4, 128), lambda i: (i, 0)),
        out_shape=jax.ShapeDtypeStruct((V // 4, 128), jnp.float32),
    )


@functools.cache
def _make_gather(N: int, B: int):
    info = plsc.get_sparse_core_info()
    nc, ns = info.num_cores, info.num_subcores
    nw = nc * ns
    n_w = N // nw
    n_blk = n_w // _BLK
    assert n_blk % 2 == 0 and n_blk >= 4
    bph = B // _BLK
    q = B // 4
    mesh = plsc.VectorSubcoreMesh(core_axis_name="c", subcore_axis_name="s")

    @functools.partial(
        pl.kernel,
        mesh=mesh,
        out_type=jax.ShapeDtypeStruct((N // 4, 128), jnp.float32),
        scratch_types=[
            pltpu.VMEM((2, _GPB, _CHUNK), jnp.int32),
            pltpu.VMEM((2, _BLK, _D), jnp.float32),
            pltpu.SemaphoreType.DMA,
            pltpu.SemaphoreType.DMA,
            pltpu.SemaphoreType.DMA,
            pltpu.SemaphoreType.DMA,
        ],
        compiler_params=pltpu.CompilerParams(use_tc_tiling_on_sc=False),
    )
    def k(idx_hbm, table_hbm, out_hbm, idx_v, rows_v, sg0, sg1, sw0, sw1):
        sem_g = (sg0, sg1)
        sem_w = (sw0, sw1)
        wid = lax.axis_index("s") * nc + lax.axis_index("c")
        kb = wid * n_blk

        def fire_block(b, p):
            irow = pl.multiple_of((kb + b) * (_BLK // _CHUNK), _GPB)
            pltpu.sync_copy(idx_hbm.at[pl.ds(irow, _GPB)], idx_v.at[p])
            for j in range(_GPB):
                pltpu.async_copy(
                    table_hbm.at[idx_v.at[p, j]],
                    rows_v.at[p, pl.ds(j * _CHUNK, _CHUNK)],
                    sem_g[p])

        def wait_gathers(p):
            pltpu.make_async_copy(
                table_hbm.at[pl.ds(0, _BLK)], rows_v.at[p], sem_g[p]).wait()

        def fire_wb(b, p):
            g = kb + b
            h = g // bph
            b0 = (g % bph) * _BLK
            j0 = b0 // q
            row = pl.multiple_of(h * (B // 4) + (b0 - j0 * q), _BLK)
            pltpu.async_copy(
                rows_v.at[p],
                out_hbm.at[pl.ds(row, _BLK), pl.ds(j0 * _D, _D)],
                sem_w[p])

        def wait_wb(p):
            pltpu.make_async_copy(
                rows_v.at[p],
                out_hbm.at[pl.ds(0, _BLK), pl.ds(0, _D)],
                sem_w[p]).wait()

        fire_block(0, 0)
        fire_block(1, 1)
        wait_gathers(0)
        fire_wb(0, 0)

        def step(i, carry):
            g = 2 * i
            for p in (0, 1):
                b = g + p
                wait_wb(p)
                fire_block(b, p)
                wait_gathers(1 - p)
                fire_wb(b - 1, 1 - p)
            return carry

        lax.fori_loop(1, n_blk // 2, step, 0)

        wait_gathers(1)
        fire_wb(n_blk - 1, 1)
        wait_wb(0)
        wait_wb(1)

    return k


@functools.cache
def _make_transpose(B: int, H: int):
    nb = B // 4

    def body(x_ref, o_ref):
        xt = x_ref[0].T
        for j in range(4):
            o_ref[0, :, j * nb:(j + 1) * nb] = xt[_D * j:_D * (j + 1), :]

    return pl.pallas_call(
        body,
        grid=(H,),
        in_specs=[pl.BlockSpec((1, nb, 128), lambda h: (h, 0, 0))],
        out_specs=pl.BlockSpec((1, _D, B), lambda h: (h, 0, 0)),
        out_shape=jax.ShapeDtypeStruct((H, _D, B), jnp.float32),
    )


def kernel(batch, table):
    B, H = batch.shape
    N = B * H
    V = table.shape[0]
    t_rm = _make_detile(V)(jnp.transpose(table))
    t_rm = t_rm.reshape(V, _D)
    idx = jnp.transpose(batch).reshape(N // _CHUNK, _CHUNK).astype(jnp.int32)
    g = _make_gather(N, B)(idx, t_rm)
    g3 = g.reshape(H, B // 4, 128)
    out_t = _make_transpose(B, H)(g3)
    return jnp.transpose(out_t, (2, 0, 1))


# speedup vs baseline: 13.1826x; 1.1178x over previous
import functools

import jax
import jax.numpy as jnp
from jax import lax
from jax.experimental import pallas as pl
from jax.experimental.pallas import tpu as pltpu
from jax.experimental.pallas import tpu_sc as plsc

_D = 32
_CHUNK = 128
_GPB = 8
_BLK = _CHUNK * _GPB


@functools.cache
def _make_detile(V: int):
    lanes = 4096
    grid = (V + lanes - 1) // lanes

    def body(x_ref, o_ref):
        r = jax.lax.broadcasted_iota(jnp.int32, (_D, 128), 0)
        c = jax.lax.broadcasted_iota(jnp.int32, (_D, 128), 1)
        q = lanes // 4
        acc = None
        for j in range(4):
            ej = jnp.where(c - _D * j == r, 1.0, 0.0).astype(jnp.float32)
            yj = jax.lax.dot_general(
                x_ref[:, q * j:q * (j + 1)], ej, (((0,), (0,)), ((), ())),
                preferred_element_type=jnp.float32)
            acc = yj if acc is None else acc + yj
        o_ref[...] = acc

    return pl.pallas_call(
        body,
        grid=(grid,),
        in_specs=[pl.BlockSpec((_D, lanes), lambda i: (0, i))],
        out_specs=pl.BlockSpec((lanes // 4, 128), lambda i: (i, 0)),
        out_shape=jax.ShapeDtypeStruct((V // 4, 128), jnp.float32),
        compiler_params=pltpu.CompilerParams(fuse_transposed_lhs_in_matmul=True),
    )


@functools.cache
def _make_gather(N: int, B: int):
    info = plsc.get_sparse_core_info()
    nc, ns = info.num_cores, info.num_subcores
    nw = nc * ns
    n_w = N // nw
    n_blk = n_w // _BLK
    assert n_blk % 2 == 0 and n_blk >= 4
    bph = B // _BLK
    q = B // 4
    mesh = plsc.VectorSubcoreMesh(core_axis_name="c", subcore_axis_name="s")

    @functools.partial(
        pl.kernel,
        mesh=mesh,
        out_type=jax.ShapeDtypeStruct((N // 4, 128), jnp.float32),
        scratch_types=[
            pltpu.VMEM((2, _GPB, _CHUNK), jnp.int32),
            pltpu.VMEM((2, _BLK, _D), jnp.float32),
            pltpu.SemaphoreType.DMA,
            pltpu.SemaphoreType.DMA,
            pltpu.SemaphoreType.DMA,
            pltpu.SemaphoreType.DMA,
        ],
        compiler_params=pltpu.CompilerParams(use_tc_tiling_on_sc=False),
    )
    def k(idx_hbm, table_hbm, out_hbm, idx_v, rows_v, sg0, sg1, sw0, sw1):
        sem_g = (sg0, sg1)
        sem_w = (sw0, sw1)
        wid = lax.axis_index("s") * nc + lax.axis_index("c")
        kb = wid * n_blk

        def fire_block(b, p):
            irow = pl.multiple_of((kb + b) * (_BLK // _CHUNK), _GPB)
            pltpu.sync_copy(idx_hbm.at[pl.ds(irow, _GPB)], idx_v.at[p])
            for j in range(_GPB):
                pltpu.async_copy(
                    table_hbm.at[idx_v.at[p, j]],
                    rows_v.at[p, pl.ds(j * _CHUNK, _CHUNK)],
                    sem_g[p])

        def wait_gathers(p):
            pltpu.make_async_copy(
                table_hbm.at[pl.ds(0, _BLK)], rows_v.at[p], sem_g[p]).wait()

        def fire_wb(b, p):
            g = kb + b
            h = g // bph
            b0 = (g % bph) * _BLK
            j0 = b0 // q
            row = pl.multiple_of(h * (B // 4) + (b0 - j0 * q), _BLK)
            pltpu.async_copy(
                rows_v.at[p],
                out_hbm.at[pl.ds(row, _BLK), pl.ds(j0 * _D, _D)],
                sem_w[p])

        def wait_wb(p):
            pltpu.make_async_copy(
                rows_v.at[p],
                out_hbm.at[pl.ds(0, _BLK), pl.ds(0, _D)],
                sem_w[p]).wait()

        fire_block(0, 0)
        fire_block(1, 1)
        wait_gathers(0)
        fire_wb(0, 0)

        def step(i, carry):
            g = 2 * i
            for p in (0, 1):
                b = g + p
                wait_wb(p)
                fire_block(b, p)
                wait_gathers(1 - p)
                fire_wb(b - 1, 1 - p)
            return carry

        lax.fori_loop(1, n_blk // 2, step, 0)

        wait_gathers(1)
        fire_wb(n_blk - 1, 1)
        wait_wb(0)
        wait_wb(1)

    return k


@functools.cache
def _make_transpose(B: int, H: int):
    nb = B // 4

    def body(x_ref, o_ref):
        xt = x_ref[0].T
        for j in range(4):
            o_ref[0, :, j * nb:(j + 1) * nb] = xt[_D * j:_D * (j + 1), :]

    return pl.pallas_call(
        body,
        grid=(H,),
        in_specs=[pl.BlockSpec((1, nb, 128), lambda h: (h, 0, 0))],
        out_specs=pl.BlockSpec((1, _D, B), lambda h: (h, 0, 0)),
        out_shape=jax.ShapeDtypeStruct((H, _D, B), jnp.float32),
    )


def kernel(batch, table):
    B, H = batch.shape
    N = B * H
    V = table.shape[0]
    t_rm = _make_detile(V)(jnp.transpose(table))
    t_rm = t_rm.reshape(V, _D)
    bt = jnp.transpose(batch).astype(jnp.int32)
    bt = (bt & -4096) | ((bt & 1023) << 2) | ((bt >> 10) & 3)
    idx = bt.reshape(N // _CHUNK, _CHUNK)
    g = _make_gather(N, B)(idx, t_rm)
    g3 = g.reshape(H, B // 4, 128)
    out_t = _make_transpose(B, H)(g3)
    return jnp.transpose(out_t, (2, 0, 1))
